# Initial kernel scaffold; baseline (speedup 1.0000x reference)
import jax
import jax.numpy as jnp
from jax.experimental import pallas as pl


def kernel(x, edge_index, edge_attr, atom_tables, bond_tables, Ws, bs, roots, ln_scale, ln_bias):
    raise NotImplementedError("write your pallas kernel here")


# speedup vs baseline: 4.2114x; 4.2114x over previous
import functools

import jax
import jax.numpy as jnp
from jax import lax
from jax.experimental import pallas as pl
from jax.experimental.pallas import tpu as pltpu, tpu_sc as plsc

N = 10000
E = 320000
D = 128
L = 3
ATOM_F = 9
ATOM_V = 100
BOND_V = 6

NC = 2
NS = 16
NW = NC * NS
CH = 128
ECHUNKS = E // CH
CH4 = 64
ECHUNKS4 = E // CH4
NCHUNKS_FULL = N // CH
NTAIL = N - NCHUNKS_FULL * CH
SUBROWS = 624
ZCH = 104
TAILROWS = N - NS * SUBROWS

_mesh = plsc.VectorSubcoreMesh(core_axis_name="c", subcore_axis_name="s")


def _wid():
    return lax.axis_index("c") * NS + lax.axis_index("s")


def _i16(ref, base):
    return ref[pl.ds(base, 16)]


@functools.partial(
    pl.kernel,
    out_type=(
        jax.ShapeDtypeStruct((NC, N, 16), jnp.float32),
        jax.ShapeDtypeStruct((N, D), jnp.float32),
        jax.ShapeDtypeStruct((E,), jnp.int32),
    ),
    mesh=_mesh,
    scratch_types=[
        pltpu.VMEM((1, CH), jnp.int32),
        pltpu.VMEM((CH,), jnp.int32),
        pltpu.VMEM((CH,), jnp.int32),
        pltpu.VMEM((CH,), jnp.int32),
        pltpu.VMEM((16,), jnp.int32),
        pltpu.VMEM((CH, 16), jnp.float32),
        pltpu.VMEM((CH, D), jnp.float32),
        pltpu.VMEM((CH, D), jnp.float32),
        pltpu.VMEM_SHARED((N, 16), jnp.float32),
        pltpu.SemaphoreType.DMA,
    ],
    compiler_params=pltpu.CompilerParams(needs_layout_passes=False),
)
def _k1_prep(row_hbm, x0, x1, x2, x3, x4, x5, x6, x7, x8, ea0, ea1, ea2,
             atom_hbm, degp_hbm, h0_hbm, eid_hbm,
             row2d, ia, ib, fidx, fidx16, ones16, acc, tmp, deg_sh, sem):
    x_hbms = (x0, x1, x2, x3, x4, x5, x6, x7, x8)
    cid = lax.axis_index("c")
    sid = lax.axis_index("s")
    wid = _wid()
    zero16 = jnp.zeros((16,), jnp.float32)

    def _z(r, _):
        ones16[r, :] = zero16
        return 0
    lax.fori_loop(0, CH, _z, 0)
    for j in range(6):
        pltpu.sync_copy(ones16.at[pl.ds(0, ZCH)],
                        deg_sh.at[pl.ds(sid * SUBROWS + j * ZCH, ZCH)])

    @pl.when(sid == NS - 1)
    def _ztail():
        pltpu.sync_copy(ones16.at[pl.ds(0, TAILROWS)],
                        deg_sh.at[pl.ds(NS * SUBROWS, TAILROWS)])
    plsc.subcore_barrier()

    one16 = jnp.ones((16,), jnp.float32)
    def _o(r, _):
        ones16[r, :] = one16
        return 0
    lax.fori_loop(0, CH, _o, 0)

    nchunks = 78 + jnp.where(wid < ECHUNKS - 78 * NW, 1, 0)

    def chunk_body(j, _):
        cidx = wid + j * NW
        base = cidx * CH
        pltpu.sync_copy(row_hbm.at[pl.ds(base, CH)], row2d.at[0])
        pltpu.sync_copy(ones16, deg_sh.at[row2d.at[0]], add=True)
        pltpu.sync_copy(ea0.at[pl.ds(base, CH)], ia)
        pltpu.sync_copy(ea1.at[pl.ds(base, CH)], ib)
        for g in range(CH // 16):
            ia[pl.ds(g * 16, 16)] = _i16(ia, g * 16) * 36 + _i16(ib, g * 16) * 6
        pltpu.sync_copy(ea2.at[pl.ds(base, CH)], ib)
        for g in range(CH // 16):
            ia[pl.ds(g * 16, 16)] = _i16(ia, g * 16) + _i16(ib, g * 16)
        pltpu.sync_copy(ia, eid_hbm.at[pl.ds(base, CH)])
        return 0

    lax.fori_loop(0, nchunks, chunk_body, 0)
    plsc.subcore_barrier()
    pltpu.sync_copy(deg_sh.at[pl.ds(sid * SUBROWS, SUBROWS)],
                    degp_hbm.at[cid, pl.ds(sid * SUBROWS, SUBROWS)])

    @pl.when(sid == NS - 1)
    def _wtail():
        pltpu.sync_copy(deg_sh.at[pl.ds(NS * SUBROWS, TAILROWS)],
                        degp_hbm.at[cid, pl.ds(NS * SUBROWS, TAILROWS)])

    def encode(base, k, fi):
        for f in range(ATOM_F):
            pltpu.sync_copy(x_hbms[f].at[pl.ds(base, k)], fi)
            for g in range(k // 16):
                fi[pl.ds(g * 16, 16)] = _i16(fi, g * 16) + f * ATOM_V
            dst = acc if k == CH else acc.at[pl.ds(0, k)]
            if f > 0:
                dst = tmp if k == CH else tmp.at[pl.ds(0, k)]
            pltpu.async_copy(atom_hbm.at[fi], dst, sem).wait()
            if f > 0:
                def _add(r, _):
                    for d in range(D // 16):
                        s = pl.ds(d * 16, 16)
                        acc[r, s] = acc[r, s] + tmp[r, s]
                    return 0
                lax.fori_loop(0, k, _add, 0)
        src = acc if k == CH else acc.at[pl.ds(0, k)]
        pltpu.sync_copy(src, h0_hbm.at[pl.ds(base, k)])

    nnode = 2 + jnp.where(wid < NCHUNKS_FULL - 2 * NW, 1, 0)

    def node_body(j, _):
        base = (wid + j * NW) * CH
        encode(base, CH, fidx)
        return 0
    lax.fori_loop(0, nnode, node_body, 0)

    @pl.when(wid == NW - 1)
    def _tail():
        encode(NCHUNKS_FULL * CH, NTAIL, fidx16)


@functools.partial(
    pl.kernel,
    out_type=jax.ShapeDtypeStruct((NC, N, D), jnp.float32),
    mesh=_mesh,
    scratch_types=[
        pltpu.VMEM((CH4,), jnp.int32),
        pltpu.VMEM((1, CH4), jnp.int32),
        pltpu.VMEM((CH4,), jnp.int32),
        pltpu.VMEM((N,), jnp.float32),
        pltpu.VMEM((216, D), jnp.float32),
        pltpu.VMEM((CH4, D), jnp.float32),
        pltpu.VMEM_SHARED((N, D), jnp.float32),
        pltpu.SemaphoreType.DMA,
    ],
    compiler_params=pltpu.CompilerParams(needs_layout_passes=False),
)
def _k4_edge(row_hbm, col_hbm, eid_hbm, hl_hbm, combo_hbm, dis_hbm,
             out_hbm, row_v, col2d, eid_v, dis_v, combo_v, hbuf, aggr_sh, sem):
    cid = lax.axis_index("c")
    sid = lax.axis_index("s")
    wid = _wid()
    iota = lax.iota(jnp.int32, 16)

    zero16 = jnp.zeros((16,), jnp.float32)
    def _z(r, _):
        for d in range(D // 16):
            hbuf[r, pl.ds(d * 16, 16)] = zero16
        return 0
    lax.fori_loop(0, CH4, _z, 0)
    for j in range(13):
        pltpu.sync_copy(hbuf.at[pl.ds(0, 48)],
                        aggr_sh.at[pl.ds(sid * SUBROWS + j * 48, 48)])

    @pl.when(sid == NS - 1)
    def _ztail():
        pltpu.sync_copy(hbuf.at[pl.ds(0, TAILROWS)],
                        aggr_sh.at[pl.ds(NS * SUBROWS, TAILROWS)])
    pltpu.sync_copy(dis_hbm, dis_v)
    pltpu.sync_copy(combo_hbm, combo_v)
    plsc.subcore_barrier()

    nchunks = (ECHUNKS4 // NW) + jnp.where(wid < ECHUNKS4 % NW, 1, 0)

    def chunk_body(j, _):
        base = (wid + j * NW) * CH4
        pltpu.sync_copy(row_hbm.at[pl.ds(base, CH4)], row_v)
        pltpu.sync_copy(col_hbm.at[pl.ds(base, CH4)], col2d.at[0])
        pltpu.sync_copy(eid_hbm.at[pl.ds(base, CH4)], eid_v)
        pltpu.async_copy(hl_hbm.at[row_v], hbuf, sem).wait()

        def group(g, _):
            rv = row_v[pl.ds(g * 16, 16)]
            cv = col2d[0, pl.ds(g * 16, 16)]
            ev = eid_v[pl.ds(g * 16, 16)]
            nrm = (plsc.load_gather(dis_v, [rv])
                   * plsc.load_gather(dis_v, [cv]))
            for i in range(16):
                nb = jnp.broadcast_to(nrm[i], (16,))
                eb = jnp.broadcast_to(ev[i], (16,))
                r = g * 16 + i
                for d in range(D // 16):
                    s = pl.ds(d * 16, 16)
                    ee = plsc.load_gather(combo_v, [eb, iota + d * 16])
                    hbuf[r, s] = nb * jnp.maximum(hbuf[r, s] + ee, 0.0)
            return 0

        lax.fori_loop(0, CH4 // 16, group, 0)
        pltpu.sync_copy(hbuf, aggr_sh.at[col2d.at[0]], add=True)
        return 0

    lax.fori_loop(0, nchunks, chunk_body, 0)
    plsc.subcore_barrier()
    pltpu.sync_copy(aggr_sh.at[pl.ds(sid * SUBROWS, SUBROWS)],
                    out_hbm.at[cid, pl.ds(sid * SUBROWS, SUBROWS)])

    @pl.when(sid == NS - 1)
    def _wtail():
        pltpu.sync_copy(aggr_sh.at[pl.ds(NS * SUBROWS, TAILROWS)],
                        out_hbm.at[cid, pl.ds(NS * SUBROWS, TAILROWS)])


def _k2_body(degp_ref, bond_ref, dis_ref, rdeg_ref, combo_ref):
    deg = degp_ref[0, :] + degp_ref[1, :] + 1.0
    dis_ref[...] = lax.rsqrt(deg)
    rdeg_ref[...] = 1.0 / deg
    b = bond_ref[...]
    combo_ref[...] = (b[:, 0][:, :, None, None, :]
                      + b[:, 1][:, None, :, None, :]
                      + b[:, 2][:, None, None, :, :])


def _tc_prep(degp, bond_tables):
    return pl.pallas_call(
        _k2_body,
        out_shape=(
            jax.ShapeDtypeStruct((N,), jnp.float32),
            jax.ShapeDtypeStruct((N,), jnp.float32),
            jax.ShapeDtypeStruct((L, BOND_V, BOND_V, BOND_V, D), jnp.float32),
        ),
    )(degp, bond_tables)


_BLK = 1000


def _k3_body(h_ref, w_ref, b_ref, out_ref):
    out_ref[...] = jnp.dot(h_ref[...], w_ref[...],
                           preferred_element_type=jnp.float32) + b_ref[...]


def _tc_matmul(h, w, b):
    return pl.pallas_call(
        _k3_body,
        grid=(N // _BLK,),
        in_specs=[
            pl.BlockSpec((_BLK, D), lambda i: (i, 0)),
            pl.BlockSpec((D, D), lambda i: (0, 0)),
            pl.BlockSpec((1, D), lambda i: (0, 0)),
        ],
        out_specs=pl.BlockSpec((_BLK, D), lambda i: (i, 0)),
        out_shape=jax.ShapeDtypeStruct((N, D), jnp.float32),
    )(h, w, b)


def _k5_body(relu, a0_ref, a1_ref, hl_ref, rdeg_ref, root_ref, sc_ref, bi_ref,
             out_ref):
    h = (a0_ref[...] + a1_ref[...]
         + jnp.maximum(hl_ref[...] + root_ref[...], 0.0) * rdeg_ref[...])
    mu = jnp.mean(h, axis=-1, keepdims=True)
    d = h - mu
    var = jnp.mean(d * d, axis=-1, keepdims=True)
    y = d * lax.rsqrt(var + 1e-5) * sc_ref[...] + bi_ref[...]
    if relu:
        y = jnp.maximum(y, 0.0)
    out_ref[...] = y


def _tc_combine(a0, a1, hl, rdeg2d, root, scale, bias, relu):
    return pl.pallas_call(
        functools.partial(_k5_body, relu),
        grid=(N // _BLK,),
        in_specs=[
            pl.BlockSpec((_BLK, D), lambda i: (i, 0)),
            pl.BlockSpec((_BLK, D), lambda i: (i, 0)),
            pl.BlockSpec((_BLK, D), lambda i: (i, 0)),
            pl.BlockSpec((_BLK, 1), lambda i: (i, 0)),
            pl.BlockSpec((1, D), lambda i: (0, 0)),
            pl.BlockSpec((1, D), lambda i: (0, 0)),
            pl.BlockSpec((1, D), lambda i: (0, 0)),
        ],
        out_specs=pl.BlockSpec((_BLK, D), lambda i: (i, 0)),
        out_shape=jax.ShapeDtypeStruct((N, D), jnp.float32),
    )(a0, a1, hl, rdeg2d, root, scale, bias)


def kernel(x, edge_index, edge_attr, atom_tables, bond_tables, Ws, bs, roots,
           ln_scale, ln_bias):
    row = edge_index[0].astype(jnp.int32)
    col = edge_index[1].astype(jnp.int32)
    xcols = [x[:, f].astype(jnp.int32) for f in range(ATOM_F)]
    eacols = [edge_attr[:, f].astype(jnp.int32) for f in range(3)]
    atom_flat = atom_tables.reshape(ATOM_F * ATOM_V, D)

    degp16, h0, eid = _k1_prep(row, *xcols, *eacols, atom_flat)
    dis, rdeg, combo = _tc_prep(degp16[:, :, 0], bond_tables)
    combo = combo.reshape(L, BOND_V ** 3, D)
    rdeg2d = rdeg.reshape(N, 1)

    h = h0
    for l in range(L):
        hl = _tc_matmul(h, Ws[l], bs[l].reshape(1, D))
        aggr = _k4_edge(row, col, eid, hl, combo[l], dis)
        h = _tc_combine(aggr[0], aggr[1], hl, rdeg2d,
                        roots[l].reshape(1, D), ln_scale[l].reshape(1, D),
                        ln_bias[l].reshape(1, D), l < L - 1)
    return h


# speedup vs baseline: 4.5439x; 1.0790x over previous
import functools

import jax
import jax.numpy as jnp
from jax import lax
from jax.experimental import pallas as pl
from jax.experimental.pallas import tpu as pltpu, tpu_sc as plsc

N = 10000
E = 320000
D = 128
L = 3
ATOM_F = 9
ATOM_V = 100
BOND_V = 6

NC = 2
NS = 16
NW = NC * NS
CH = 128
ECHUNKS = E // CH
CH4 = 64
E4 = 158 * CH4 * NW
NCH4 = E4 // (CH4 * NW)
NCHUNKS_FULL = N // CH
NTAIL = N - NCHUNKS_FULL * CH
SUBROWS = 624
ZCH = 104
TAILROWS = N - NS * SUBROWS

_mesh = plsc.VectorSubcoreMesh(core_axis_name="c", subcore_axis_name="s")


def _wid():
    return lax.axis_index("c") * NS + lax.axis_index("s")


def _i16(ref, base):
    return ref[pl.ds(base, 16)]


@functools.partial(
    pl.kernel,
    out_type=(
        jax.ShapeDtypeStruct((NC, N, 16), jnp.float32),
        jax.ShapeDtypeStruct((N, D), jnp.float32),
        jax.ShapeDtypeStruct((E,), jnp.int32),
    ),
    mesh=_mesh,
    scratch_types=[
        pltpu.VMEM((1, CH), jnp.int32),
        pltpu.VMEM((CH,), jnp.int32),
        pltpu.VMEM((CH,), jnp.int32),
        pltpu.VMEM((CH,), jnp.int32),
        pltpu.VMEM((16,), jnp.int32),
        pltpu.VMEM((CH, 16), jnp.float32),
        pltpu.VMEM((CH, D), jnp.float32),
        pltpu.VMEM((CH, D), jnp.float32),
        pltpu.VMEM_SHARED((N, 16), jnp.float32),
        pltpu.SemaphoreType.DMA,
    ],
    compiler_params=pltpu.CompilerParams(needs_layout_passes=False),
)
def _k1_prep(row_hbm, x0, x1, x2, x3, x4, x5, x6, x7, x8, ea0, ea1, ea2,
             atom_hbm, degp_hbm, h0_hbm, eid_hbm,
             row2d, ia, ib, fidx, fidx16, ones16, acc, tmp, deg_sh, sem):
    x_hbms = (x0, x1, x2, x3, x4, x5, x6, x7, x8)
    cid = lax.axis_index("c")
    sid = lax.axis_index("s")
    wid = _wid()
    zero16 = jnp.zeros((16,), jnp.float32)

    def _z(r, _):
        ones16[r, :] = zero16
        return 0
    lax.fori_loop(0, CH, _z, 0)
    for j in range(6):
        pltpu.sync_copy(ones16.at[pl.ds(0, ZCH)],
                        deg_sh.at[pl.ds(sid * SUBROWS + j * ZCH, ZCH)])

    @pl.when(sid == NS - 1)
    def _ztail():
        pltpu.sync_copy(ones16.at[pl.ds(0, TAILROWS)],
                        deg_sh.at[pl.ds(NS * SUBROWS, TAILROWS)])
    plsc.subcore_barrier()

    one16 = jnp.ones((16,), jnp.float32)
    def _o(r, _):
        ones16[r, :] = one16
        return 0
    lax.fori_loop(0, CH, _o, 0)

    nchunks = 78 + jnp.where(wid < ECHUNKS - 78 * NW, 1, 0)

    def chunk_body(j, _):
        cidx = wid + j * NW
        base = cidx * CH
        pltpu.sync_copy(row_hbm.at[pl.ds(base, CH)], row2d.at[0])
        pltpu.sync_copy(ones16, deg_sh.at[row2d.at[0]], add=True)
        pltpu.sync_copy(ea0.at[pl.ds(base, CH)], ia)
        pltpu.sync_copy(ea1.at[pl.ds(base, CH)], ib)
        for g in range(CH // 16):
            ia[pl.ds(g * 16, 16)] = _i16(ia, g * 16) * 36 + _i16(ib, g * 16) * 6
        pltpu.sync_copy(ea2.at[pl.ds(base, CH)], ib)
        for g in range(CH // 16):
            ia[pl.ds(g * 16, 16)] = _i16(ia, g * 16) + _i16(ib, g * 16)
        pltpu.sync_copy(ia, eid_hbm.at[pl.ds(base, CH)])
        return 0

    lax.fori_loop(0, nchunks, chunk_body, 0)
    plsc.subcore_barrier()
    pltpu.sync_copy(deg_sh.at[pl.ds(sid * SUBROWS, SUBROWS)],
                    degp_hbm.at[cid, pl.ds(sid * SUBROWS, SUBROWS)])

    @pl.when(sid == NS - 1)
    def _wtail():
        pltpu.sync_copy(deg_sh.at[pl.ds(NS * SUBROWS, TAILROWS)],
                        degp_hbm.at[cid, pl.ds(NS * SUBROWS, TAILROWS)])

    def encode(base, k, fi):
        for f in range(ATOM_F):
            pltpu.sync_copy(x_hbms[f].at[pl.ds(base, k)], fi)
            for g in range(k // 16):
                fi[pl.ds(g * 16, 16)] = _i16(fi, g * 16) + f * ATOM_V
            dst = acc if k == CH else acc.at[pl.ds(0, k)]
            if f > 0:
                dst = tmp if k == CH else tmp.at[pl.ds(0, k)]
            pltpu.async_copy(atom_hbm.at[fi], dst, sem).wait()
            if f > 0:
                def _add(r, _):
                    for d in range(D // 16):
                        s = pl.ds(d * 16, 16)
                        acc[r, s] = acc[r, s] + tmp[r, s]
                    return 0
                lax.fori_loop(0, k, _add, 0)
        src = acc if k == CH else acc.at[pl.ds(0, k)]
        pltpu.sync_copy(src, h0_hbm.at[pl.ds(base, k)])

    nnode = 2 + jnp.where(wid < NCHUNKS_FULL - 2 * NW, 1, 0)

    def node_body(j, _):
        base = (wid + j * NW) * CH
        encode(base, CH, fidx)
        return 0
    lax.fori_loop(0, nnode, node_body, 0)

    @pl.when(wid == NW - 1)
    def _tail():
        encode(NCHUNKS_FULL * CH, NTAIL, fidx16)


@functools.partial(
    pl.kernel,
    out_type=jax.ShapeDtypeStruct((E,), jnp.float32),
    mesh=_mesh,
    scratch_types=[
        pltpu.VMEM((N,), jnp.float32),
        pltpu.VMEM((CH,), jnp.int32),
        pltpu.VMEM((CH,), jnp.int32),
        pltpu.VMEM((CH,), jnp.float32),
    ],
    compiler_params=pltpu.CompilerParams(needs_layout_passes=False),
)
def _k1b_norm(row_hbm, col_hbm, dis_hbm, out_hbm, dis_v, r_v, c_v, n_v):
    wid = _wid()
    pltpu.sync_copy(dis_hbm, dis_v)
    nchunks = (ECHUNKS // NW) + jnp.where(wid < ECHUNKS % NW, 1, 0)

    def chunk_body(j, _):
        base = (wid + j * NW) * CH
        pltpu.sync_copy(row_hbm.at[pl.ds(base, CH)], r_v)
        pltpu.sync_copy(col_hbm.at[pl.ds(base, CH)], c_v)
        for g in range(CH // 16):
            s = pl.ds(g * 16, 16)
            n_v[s] = (plsc.load_gather(dis_v, [r_v[s]])
                      * plsc.load_gather(dis_v, [c_v[s]]))
        pltpu.sync_copy(n_v, out_hbm.at[pl.ds(base, CH)])
        return 0

    lax.fori_loop(0, nchunks, chunk_body, 0)


@functools.partial(
    pl.kernel,
    out_type=jax.ShapeDtypeStruct((NC, N, D), jnp.float32),
    mesh=_mesh,
    scratch_types=[
        pltpu.VMEM((CH4,), jnp.int32),
        pltpu.VMEM((CH4,), jnp.int32),
        pltpu.VMEM((1, CH4), jnp.int32),
        pltpu.VMEM((1, CH4), jnp.int32),
        pltpu.VMEM((1, CH4), jnp.int32),
        pltpu.VMEM((1, CH4), jnp.int32),
        pltpu.VMEM((CH4,), jnp.int32),
        pltpu.VMEM((CH4,), jnp.int32),
        pltpu.VMEM((CH4,), jnp.float32),
        pltpu.VMEM((CH4,), jnp.float32),
        pltpu.VMEM((216, D), jnp.float32),
        pltpu.VMEM((CH4, D), jnp.float32),
        pltpu.VMEM((CH4, D), jnp.float32),
        pltpu.VMEM_SHARED((N, D), jnp.float32),
        pltpu.SemaphoreType.DMA,
        pltpu.SemaphoreType.DMA,
        pltpu.SemaphoreType.DMA,
    ],
    compiler_params=pltpu.CompilerParams(needs_layout_passes=False),
)
def _k4_edge(row_hbm, col_hbm, eid_hbm, nrm_hbm, hl_hbm, combo_hbm,
             out_hbm, row0, row1, col0, col1, cs0, cs1, eid0, eid1,
             nv0, nv1, combo_v, hb0, hb1, aggr_sh, sem_g, sem_s, sem_i):
    cid = lax.axis_index("c")
    sid = lax.axis_index("s")
    wid = _wid()
    iota = lax.iota(jnp.int32, 16)
    rows = (row0, row1)
    cols = (col0, col1)
    css = (cs0, cs1)
    eids = (eid0, eid1)
    nvs = (nv0, nv1)
    hbs = (hb0, hb1)

    zero16 = jnp.zeros((16,), jnp.float32)
    def _z(r, _):
        for d in range(D // 16):
            hb0[r, pl.ds(d * 16, 16)] = zero16
        return 0
    lax.fori_loop(0, CH4, _z, 0)
    for j in range(13):
        pltpu.sync_copy(hb0.at[pl.ds(0, 48)],
                        aggr_sh.at[pl.ds(sid * SUBROWS + j * 48, 48)])

    @pl.when(sid == NS - 1)
    def _ztail():
        pltpu.sync_copy(hb0.at[pl.ds(0, TAILROWS)],
                        aggr_sh.at[pl.ds(NS * SUBROWS, TAILROWS)])
    pltpu.sync_copy(combo_hbm, combo_v)
    plsc.subcore_barrier()

    def cbase(j):
        return (wid + j * NW) * CH4

    def issue_idx(j, s):
        b = cbase(j)
        pltpu.async_copy(row_hbm.at[pl.ds(b, CH4)], rows[s], sem_i)
        pltpu.async_copy(col_hbm.at[pl.ds(b, CH4)], cols[s].at[0], sem_i)
        pltpu.async_copy(eid_hbm.at[pl.ds(b, CH4)], eids[s], sem_i)
        pltpu.async_copy(nrm_hbm.at[pl.ds(b, CH4)], nvs[s], sem_i)

    def drain_idx(s):
        pltpu.make_async_copy(row_hbm.at[pl.ds(0, CH4)], rows[s], sem_i).wait()
        pltpu.make_async_copy(col_hbm.at[pl.ds(0, CH4)], cols[s].at[0],
                              sem_i).wait()
        pltpu.make_async_copy(eid_hbm.at[pl.ds(0, CH4)], eids[s], sem_i).wait()
        pltpu.make_async_copy(nrm_hbm.at[pl.ds(0, CH4)], nvs[s], sem_i).wait()

    def drain_rows(sem, s):
        pltpu.make_async_copy(hl_hbm.at[pl.ds(0, CH4)], hbs[s], sem).wait()

    def compute(s):
        hbuf, nv_, ev_ = hbs[s], nvs[s], eids[s]
        for q in range(CH4 // 16):
            css[s][0, pl.ds(q * 16, 16)] = cols[s][0, pl.ds(q * 16, 16)]

        def group(g, _):
            nrm = nv_[pl.ds(g * 16, 16)]
            ev = ev_[pl.ds(g * 16, 16)]
            for i in range(16):
                nb = jnp.broadcast_to(nrm[i], (16,))
                eb = jnp.broadcast_to(ev[i], (16,))
                r = g * 16 + i
                for d in range(D // 16):
                    sl = pl.ds(d * 16, 16)
                    ee = plsc.load_gather(combo_v, [eb, iota + d * 16])
                    hbuf[r, sl] = nb * jnp.maximum(hbuf[r, sl] + ee, 0.0)
            return 0

        lax.fori_loop(0, CH4 // 16, group, 0)

    def body(j, s, first, issue2):
        ns = 1 - s
        drain_rows(sem_g, s)
        compute(s)
        drain_idx(ns)
        if not first:
            drain_rows(sem_s, ns)
        pltpu.async_copy(hl_hbm.at[rows[ns]], hbs[ns], sem_g)
        pltpu.async_copy(hbs[s], aggr_sh.at[css[s].at[0]], sem_s, add=True)
        if issue2:
            issue_idx(j + 2, s)

    b0 = cbase(0)
    pltpu.sync_copy(row_hbm.at[pl.ds(b0, CH4)], row0)
    pltpu.sync_copy(col_hbm.at[pl.ds(b0, CH4)], col0.at[0])
    pltpu.sync_copy(eid_hbm.at[pl.ds(b0, CH4)], eid0)
    pltpu.sync_copy(nrm_hbm.at[pl.ds(b0, CH4)], nv0)
    pltpu.async_copy(hl_hbm.at[row0], hb0, sem_g)
    issue_idx(1, 1)

    body(jnp.int32(0), 0, True, True)
    body(jnp.int32(1), 1, False, True)

    def pair(p, _):
        body(2 * p, 0, False, True)
        body(2 * p + 1, 1, False, True)
        return 0
    lax.fori_loop(1, NCH4 // 2 - 1, pair, 0)

    jl = NCH4 - 2
    drain_rows(sem_g, 0)
    compute(0)
    drain_idx(1)
    drain_rows(sem_s, 1)
    pltpu.async_copy(hl_hbm.at[rows[1]], hbs[1], sem_g)
    pltpu.async_copy(hbs[0], aggr_sh.at[css[0].at[0]], sem_s, add=True)
    drain_rows(sem_g, 1)
    compute(1)
    pltpu.async_copy(hbs[1], aggr_sh.at[css[1].at[0]], sem_s, add=True)
    drain_rows(sem_s, 0)
    drain_rows(sem_s, 1)
    del jl

    plsc.subcore_barrier()
    pltpu.sync_copy(aggr_sh.at[pl.ds(sid * SUBROWS, SUBROWS)],
                    out_hbm.at[cid, pl.ds(sid * SUBROWS, SUBROWS)])

    @pl.when(sid == NS - 1)
    def _wtail():
        pltpu.sync_copy(aggr_sh.at[pl.ds(NS * SUBROWS, TAILROWS)],
                        out_hbm.at[cid, pl.ds(NS * SUBROWS, TAILROWS)])


def _k2_body(degp_ref, bond_ref, dis_ref, rdeg_ref, combo_ref):
    deg = degp_ref[0, :] + degp_ref[1, :] + 1.0
    dis_ref[...] = lax.rsqrt(deg)
    rdeg_ref[...] = 1.0 / deg
    b = bond_ref[...]
    combo_ref[...] = (b[:, 0][:, :, None, None, :]
                      + b[:, 1][:, None, :, None, :]
                      + b[:, 2][:, None, None, :, :])


def _tc_prep(degp, bond_tables):
    return pl.pallas_call(
        _k2_body,
        out_shape=(
            jax.ShapeDtypeStruct((N,), jnp.float32),
            jax.ShapeDtypeStruct((N,), jnp.float32),
            jax.ShapeDtypeStruct((L, BOND_V, BOND_V, BOND_V, D), jnp.float32),
        ),
    )(degp, bond_tables)


_BLK = 1000


def _k3_body(h_ref, w_ref, b_ref, out_ref):
    out_ref[...] = jnp.dot(h_ref[...], w_ref[...],
                           preferred_element_type=jnp.float32) + b_ref[...]


def _tc_matmul(h, w, b):
    return pl.pallas_call(
        _k3_body,
        grid=(N // _BLK,),
        in_specs=[
            pl.BlockSpec((_BLK, D), lambda i: (i, 0)),
            pl.BlockSpec((D, D), lambda i: (0, 0)),
            pl.BlockSpec((1, D), lambda i: (0, 0)),
        ],
        out_specs=pl.BlockSpec((_BLK, D), lambda i: (i, 0)),
        out_shape=jax.ShapeDtypeStruct((N, D), jnp.float32),
    )(h, w, b)


def _k5_body(relu, a0_ref, a1_ref, hl_ref, rdeg_ref, root_ref, sc_ref, bi_ref,
             out_ref):
    h = (a0_ref[...] + a1_ref[...]
         + jnp.maximum(hl_ref[...] + root_ref[...], 0.0) * rdeg_ref[...])
    mu = jnp.mean(h, axis=-1, keepdims=True)
    d = h - mu
    var = jnp.mean(d * d, axis=-1, keepdims=True)
    y = d * lax.rsqrt(var + 1e-5) * sc_ref[...] + bi_ref[...]
    if relu:
        y = jnp.maximum(y, 0.0)
    out_ref[...] = y


def _tc_combine(a0, a1, hl, rdeg2d, root, scale, bias, relu):
    return pl.pallas_call(
        functools.partial(_k5_body, relu),
        grid=(N // _BLK,),
        in_specs=[
            pl.BlockSpec((_BLK, D), lambda i: (i, 0)),
            pl.BlockSpec((_BLK, D), lambda i: (i, 0)),
            pl.BlockSpec((_BLK, D), lambda i: (i, 0)),
            pl.BlockSpec((_BLK, 1), lambda i: (i, 0)),
            pl.BlockSpec((1, D), lambda i: (0, 0)),
            pl.BlockSpec((1, D), lambda i: (0, 0)),
            pl.BlockSpec((1, D), lambda i: (0, 0)),
        ],
        out_specs=pl.BlockSpec((_BLK, D), lambda i: (i, 0)),
        out_shape=jax.ShapeDtypeStruct((N, D), jnp.float32),
    )(a0, a1, hl, rdeg2d, root, scale, bias)


def kernel(x, edge_index, edge_attr, atom_tables, bond_tables, Ws, bs, roots,
           ln_scale, ln_bias):
    row = edge_index[0].astype(jnp.int32)
    col = edge_index[1].astype(jnp.int32)
    xcols = [x[:, f].astype(jnp.int32) for f in range(ATOM_F)]
    eacols = [edge_attr[:, f].astype(jnp.int32) for f in range(3)]
    atom_flat = atom_tables.reshape(ATOM_F * ATOM_V, D)

    degp16, h0, eid = _k1_prep(row, *xcols, *eacols, atom_flat)
    dis, rdeg, combo = _tc_prep(degp16[:, :, 0], bond_tables)
    combo = combo.reshape(L, BOND_V ** 3, D)
    rdeg2d = rdeg.reshape(N, 1)
    nrm = _k1b_norm(row, col, dis)

    pad = E4 - E
    zi = jnp.zeros((pad,), jnp.int32)
    row4 = jnp.concatenate([row, zi])
    col4 = jnp.concatenate([col, zi])
    eid4 = jnp.concatenate([eid, zi])
    nrm4 = jnp.concatenate([nrm, jnp.zeros((pad,), jnp.float32)])

    h = h0
    for l in range(L):
        hl = _tc_matmul(h, Ws[l], bs[l].reshape(1, D))
        aggr = _k4_edge(row4, col4, eid4, nrm4, hl, combo[l])
        h = _tc_combine(aggr[0], aggr[1], hl, rdeg2d,
                        roots[l].reshape(1, D), ln_scale[l].reshape(1, D),
                        ln_bias[l].reshape(1, D), l < L - 1)
    return h
